# baseline (speedup 1.0000x reference)
import functools

import jax
import jax.numpy as jnp
from jax import lax
from jax.experimental import pallas as pl
from jax.experimental.pallas import tpu as pltpu
from jax.experimental.pallas import tpu_sc as plsc

B, N, K, DIM, HID = 2, 4096, 16, 256, 512
DIM2 = 2 * DIM
XP = 128
DIMP = DIM + XP
BN = 256
NB = N // BN
BM = 1024
CH = 128
NSC, NSUB = 2, 16
NW = NSC * NSUB
CPW = N // CH


def _fold_body(wq, wa1, wkv, wp2, wqa_ref, wg_ref, wp2cat_ref):
    a1 = wa1[...]
    wqa_ref[...] = jnp.dot(wq[...], a1, preferred_element_type=jnp.float32)
    wg_ref[:, :DIM] = jnp.dot(wkv[:, :DIM], a1, preferred_element_type=jnp.float32)
    wg_ref[:, DIM:] = wkv[:, DIM:]
    wp2cat_ref[:, :DIM] = wp2[...]
    wp2cat_ref[:, DIM:] = jnp.dot(wp2[...], a1, preferred_element_type=jnp.float32)


def _rne16(x):
    u = lax.bitcast_convert_type(x, jnp.int32)
    return (u + 0x7FFF + ((u >> 16) & 1)) >> 16


def _pre_body(feats, xyzp, g1b1, wqa, wg, qa_ref, g_ref):
    x = feats[...]
    m = jnp.mean(x, axis=-1, keepdims=True)
    xc = x - m
    v = jnp.mean(xc * xc, axis=-1, keepdims=True)
    xn = xc * lax.rsqrt(v + 1e-5) * g1b1[0:1, :] + g1b1[1:2, :]
    qa_ref[...] = jnp.dot(xn, wqa[...], preferred_element_type=jnp.float32)
    kav = jnp.dot(xn, wg[...], preferred_element_type=jnp.float32)
    ka_b = _rne16(kav[:, :DIM]) & 0xFFFF
    v_b = _rne16(kav[:, DIM:])
    g_ref[:, :DIM] = ka_b | (v_b << 16)
    g_ref[:, DIM:] = lax.bitcast_convert_type(xyzp[...], jnp.int32)


def _gather_body(gtab, idxf, gg_out, idx_v, gbuf0, gbuf1, sem_g0, sem_g1,
                 sem_w0, sem_w1):
    wid = lax.axis_index("s") * NSC + lax.axis_index("c")
    pltpu.sync_copy(idxf.at[wid], idx_v)
    gbufs = (gbuf0, gbuf1)
    gsems = (sem_g0, sem_g1)
    wsems = (sem_w0, sem_w1)

    def start_gather(c):
        isl = idx_v.at[pl.ds(c * CH, CH)]
        return pltpu.async_copy(gtab.at[isl], gbufs[c % 2], gsems[c % 2])

    gathers = {0: start_gather(0)}
    writes = {}
    for c in range(CPW):
        b = c % 2
        if c + 1 < CPW:
            if c >= 1:
                writes[c - 1].wait()
            gathers[c + 1] = start_gather(c + 1)
        gathers[c].wait()
        writes[c] = pltpu.async_copy(gbufs[b], gg_out.at[wid, pl.ds(c * CH, CH)],
                                     wsems[b])
    writes[CPW - 2].wait()
    writes[CPW - 1].wait()


def _attn_body(feats, xyzp, qa, gg, wp1p, wp2cat, wa2, wproj, wf1, wf2,
               g2b2, out_ref):
    bf = jnp.bfloat16
    f = feats[...]
    xp = xyzp[...]
    q = qa[...]
    w1b = wp1p[...].astype(bf)
    w2b = wp2cat[...].astype(bf)
    wab = wa2[...].astype(bf)
    m = jnp.full((BN, DIM), -1e30, jnp.float32)
    s = jnp.zeros((BN, DIM), jnp.float32)
    acc = jnp.zeros((BN, DIM), jnp.float32)
    for k in range(K):
        slab = gg[0, k]
        kav = slab[:, :DIM]
        ka = lax.bitcast_convert_type(kav << 16, jnp.float32)
        v = lax.bitcast_convert_type(kav & jnp.int32(-65536), jnp.float32)
        rel = lax.bitcast_convert_type(slab[:, DIM:], jnp.float32) - xp
        h = jnp.maximum(jnp.dot(rel.astype(bf), w1b,
                                preferred_element_type=jnp.float32), 0.0)
        pp = jnp.dot(h.astype(bf), w2b, preferred_element_type=jnp.float32)
        t = jnp.dot(jnp.maximum(q - ka + pp[:, DIM:], 0.0).astype(bf), wab,
                    preferred_element_type=jnp.float32)
        mn = jnp.maximum(m, t)
        sc = jnp.exp(m - mn)
        e = jnp.exp(t - mn)
        s = s * sc + e
        acc = acc * sc + e * (v + pp[:, :DIM])
        m = mn
    out = acc / s
    y = f + jnp.dot(out.astype(bf), wproj[...].astype(bf),
                    preferred_element_type=jnp.float32)
    mu = jnp.mean(y, axis=-1, keepdims=True)
    yc = y - mu
    var = jnp.mean(yc * yc, axis=-1, keepdims=True)
    ln = yc * lax.rsqrt(var + 1e-5) * g2b2[0:1, :] + g2b2[1:2, :]
    z = y + jnp.dot(
        jnp.maximum(jnp.dot(ln.astype(bf), wf1[...].astype(bf),
                            preferred_element_type=jnp.float32), 0.0).astype(bf),
        wf2[...].astype(bf), preferred_element_type=jnp.float32)
    out_ref[...] = z


def _fold_call(wq, wa1, wkv, wp2):
    return pl.pallas_call(
        _fold_body,
        out_shape=(
            jax.ShapeDtypeStruct((DIM, DIM), jnp.float32),
            jax.ShapeDtypeStruct((DIM, DIM2), jnp.float32),
            jax.ShapeDtypeStruct((DIM, DIM2), jnp.float32),
        ),
    )(wq, wa1, wkv, wp2)


def _pre_call(feats2, xyzp, g1b1, wqa, wg):
    nblk = (B * N) // BM
    return pl.pallas_call(
        _pre_body,
        grid=(nblk,),
        in_specs=[
            pl.BlockSpec((BM, DIM), lambda i: (i, 0)),
            pl.BlockSpec((BM, XP), lambda i: (i, 0)),
            pl.BlockSpec((2, DIM), lambda i: (0, 0)),
            pl.BlockSpec((DIM, DIM), lambda i: (0, 0)),
            pl.BlockSpec((DIM, DIM2), lambda i: (0, 0)),
        ],
        out_specs=(
            pl.BlockSpec((BM, DIM), lambda i: (i, 0)),
            pl.BlockSpec((BM, DIMP), lambda i: (i, 0)),
        ),
        out_shape=(
            jax.ShapeDtypeStruct((B * N, DIM), jnp.float32),
            jax.ShapeDtypeStruct((B * N, DIMP), jnp.int32),
        ),
    )(feats2, xyzp, g1b1, wqa, wg)


def _gather_call(gtab, idxf):
    k = functools.partial(
        pl.kernel,
        mesh=plsc.VectorSubcoreMesh(core_axis_name="c", subcore_axis_name="s"),
        out_type=jax.ShapeDtypeStruct((B * K, N, DIMP), jnp.int32),
        scratch_types=[
            pltpu.VMEM((N,), jnp.int32),
            pltpu.VMEM((CH, DIMP), jnp.int32),
            pltpu.VMEM((CH, DIMP), jnp.int32),
            pltpu.SemaphoreType.DMA,
            pltpu.SemaphoreType.DMA,
            pltpu.SemaphoreType.DMA,
            pltpu.SemaphoreType.DMA,
        ],
    )(_gather_body)
    return k(gtab, idxf)


def _attn_call(feats2, xyzp, qa, gg, wp1p, wp2cat, wa2, wproj, wf1, wf2, g2b2):
    return pl.pallas_call(
        _attn_body,
        grid=(B * NB,),
        in_specs=[
            pl.BlockSpec((BN, DIM), lambda i: (i, 0)),
            pl.BlockSpec((BN, XP), lambda i: (i, 0)),
            pl.BlockSpec((BN, DIM), lambda i: (i, 0)),
            pl.BlockSpec((1, K, BN, DIMP), lambda i: (i // NB, 0, i % NB, 0)),
            pl.BlockSpec((XP, DIM), lambda i: (0, 0)),
            pl.BlockSpec((DIM, DIM2), lambda i: (0, 0)),
            pl.BlockSpec((DIM, DIM), lambda i: (0, 0)),
            pl.BlockSpec((DIM, DIM), lambda i: (0, 0)),
            pl.BlockSpec((DIM, HID), lambda i: (0, 0)),
            pl.BlockSpec((HID, DIM), lambda i: (0, 0)),
            pl.BlockSpec((2, DIM), lambda i: (0, 0)),
        ],
        out_specs=pl.BlockSpec((BN, DIM), lambda i: (i, 0)),
        out_shape=jax.ShapeDtypeStruct((B * N, DIM), jnp.float32),
    )(feats2, xyzp, qa, gg, wp1p, wp2cat, wa2, wproj, wf1, wf2, g2b2)


def kernel(xyz, feats, idx, g1, b1, g2, b2, Wq, Wkv, Wp1, Wp2, Wa1, Wa2, Wproj, Wf1, Wf2):
    feats2 = feats.reshape(B * N, DIM)
    xyzp = jnp.pad(xyz, ((0, 0), (0, 0), (0, XP - 3))).reshape(B * N, XP)
    wp1p = jnp.pad(Wp1, ((0, XP - 3), (0, 0)))
    g1b1 = jnp.stack([g1, b1])
    g2b2 = jnp.stack([g2, b2])
    idxf = (idx + (jnp.arange(B, dtype=jnp.int32) * N)[:, None, None]
            ).transpose(0, 2, 1).reshape(B * K, N)

    wqa, wg, wp2cat = _fold_call(Wq, Wa1, Wkv, Wp2)
    qa, gtab = _pre_call(feats2, xyzp, g1b1, wqa, wg)
    gg = _gather_call(gtab, idxf)
    z = _attn_call(feats2, xyzp, qa, gg.reshape(B, K, N, DIMP),
                   wp1p, wp2cat, Wa2, Wproj, Wf1, Wf2, g2b2)
    return z.reshape(B, N, DIM)


# speedup vs baseline: 1.0859x; 1.0859x over previous
import functools

import jax
import jax.numpy as jnp
from jax import lax
from jax.experimental import pallas as pl
from jax.experimental.pallas import tpu as pltpu
from jax.experimental.pallas import tpu_sc as plsc

B, N, K, DIM, HID = 2, 4096, 16, 256, 512
DIM2 = 2 * DIM
XP = 128
DIMP = DIM + XP
BN = 256
NB = N // BN
BM = 1024
CH = 128
NSC, NSUB = 2, 16
NW = NSC * NSUB
NPW = (K * N) // NW
CPW = NPW // CH


def _fold_body(wq, wa1, wkv, wp2, wqa_ref, wg_ref, wp2cat_ref):
    a1 = wa1[...]
    wqa_ref[...] = jnp.dot(wq[...], a1, preferred_element_type=jnp.float32)
    wg_ref[:, :DIM] = jnp.dot(wkv[:, :DIM], a1, preferred_element_type=jnp.float32)
    wg_ref[:, DIM:] = wkv[:, DIM:]
    wp2cat_ref[:, :DIM] = wp2[...]
    wp2cat_ref[:, DIM:] = jnp.dot(wp2[...], a1, preferred_element_type=jnp.float32)


def _rne16(x):
    u = lax.bitcast_convert_type(x, jnp.int32)
    return (u + 0x7FFF + ((u >> 16) & 1)) >> 16


def _pre_body(feats, xyzp, g1b1, wqa, wg, qa_ref, g_ref):
    x = feats[...]
    m = jnp.mean(x, axis=-1, keepdims=True)
    xc = x - m
    v = jnp.mean(xc * xc, axis=-1, keepdims=True)
    xn = xc * lax.rsqrt(v + 1e-5) * g1b1[0:1, :] + g1b1[1:2, :]
    qa_ref[...] = jnp.dot(xn, wqa[...], preferred_element_type=jnp.float32)
    kav = jnp.dot(xn, wg[...], preferred_element_type=jnp.float32)
    ka_b = _rne16(kav[:, :DIM]) & 0xFFFF
    v_b = _rne16(kav[:, DIM:])
    g_ref[:, :DIM] = ka_b | (v_b << 16)
    g_ref[:, DIM:] = lax.bitcast_convert_type(xyzp[...], jnp.int32)


def _gather_body(gtab, idxf, gg_out, idx_v, gbuf0, gbuf1, sem_g0, sem_g1,
                 sem_w0, sem_w1):
    wid = lax.axis_index("s") * NSC + lax.axis_index("c")
    pltpu.sync_copy(idxf.at[wid], idx_v)
    gbufs = (gbuf0, gbuf1)
    gsems = (sem_g0, sem_g1)
    wsems = (sem_w0, sem_w1)

    def start_gather(c):
        isl = idx_v.at[pl.ds(c * CH, CH)]
        return pltpu.async_copy(gtab.at[isl], gbufs[c % 2], gsems[c % 2])

    gathers = {0: start_gather(0)}
    writes = {}
    for c in range(CPW):
        b = c % 2
        if c + 1 < CPW:
            if c >= 1:
                writes[c - 1].wait()
            gathers[c + 1] = start_gather(c + 1)
        gathers[c].wait()
        writes[c] = pltpu.async_copy(gbufs[b], gg_out.at[wid, pl.ds(c * CH, CH)],
                                     wsems[b])
    writes[CPW - 2].wait()
    writes[CPW - 1].wait()


def _attn_body(feats, xyzp, qa, gg, wp1p, wp2cat, wa2, wproj, wf1, wf2,
               g2b2, out_ref):
    bf = jnp.bfloat16
    f = feats[...]
    xp = xyzp[...]
    q = qa[...]
    w1b = wp1p[...].astype(bf)
    w2b = wp2cat[...].astype(bf)
    wab = wa2[...].astype(bf)
    m = jnp.full((BN, DIM), -1e30, jnp.float32)
    s = jnp.zeros((BN, DIM), jnp.float32)
    acc = jnp.zeros((BN, DIM), jnp.float32)
    for k in range(K):
        slab = gg[0, k]
        kav = slab[:, :DIM]
        ka = lax.bitcast_convert_type(kav << 16, jnp.float32)
        v = lax.bitcast_convert_type(kav & jnp.int32(-65536), jnp.float32)
        rel = lax.bitcast_convert_type(slab[:, DIM:], jnp.float32) - xp
        h = jnp.maximum(jnp.dot(rel.astype(bf), w1b,
                                preferred_element_type=jnp.float32), 0.0)
        pp = jnp.dot(h.astype(bf), w2b, preferred_element_type=jnp.float32)
        t = jnp.dot(jnp.maximum(q - ka + pp[:, DIM:], 0.0).astype(bf), wab,
                    preferred_element_type=jnp.float32)
        mn = jnp.maximum(m, t)
        sc = jnp.exp(m - mn)
        e = jnp.exp(t - mn)
        s = s * sc + e
        acc = acc * sc + e * (v + pp[:, :DIM])
        m = mn
    out = acc / s
    y = f + jnp.dot(out.astype(bf), wproj[...].astype(bf),
                    preferred_element_type=jnp.float32)
    mu = jnp.mean(y, axis=-1, keepdims=True)
    yc = y - mu
    var = jnp.mean(yc * yc, axis=-1, keepdims=True)
    ln = yc * lax.rsqrt(var + 1e-5) * g2b2[0:1, :] + g2b2[1:2, :]
    z = y + jnp.dot(
        jnp.maximum(jnp.dot(ln.astype(bf), wf1[...].astype(bf),
                            preferred_element_type=jnp.float32), 0.0).astype(bf),
        wf2[...].astype(bf), preferred_element_type=jnp.float32)
    out_ref[...] = z


def _fold_call(wq, wa1, wkv, wp2):
    return pl.pallas_call(
        _fold_body,
        out_shape=(
            jax.ShapeDtypeStruct((DIM, DIM), jnp.float32),
            jax.ShapeDtypeStruct((DIM, DIM2), jnp.float32),
            jax.ShapeDtypeStruct((DIM, DIM2), jnp.float32),
        ),
    )(wq, wa1, wkv, wp2)


def _pre_call(feats2, xyzp, g1b1, wqa, wg):
    nblk = (B * N) // BM
    return pl.pallas_call(
        _pre_body,
        grid=(nblk,),
        in_specs=[
            pl.BlockSpec((BM, DIM), lambda i: (i, 0)),
            pl.BlockSpec((BM, XP), lambda i: (i, 0)),
            pl.BlockSpec((2, DIM), lambda i: (0, 0)),
            pl.BlockSpec((DIM, DIM), lambda i: (0, 0)),
            pl.BlockSpec((DIM, DIM2), lambda i: (0, 0)),
        ],
        out_specs=(
            pl.BlockSpec((BM, DIM), lambda i: (i, 0)),
            pl.BlockSpec((BM, DIMP), lambda i: (i, 0)),
        ),
        out_shape=(
            jax.ShapeDtypeStruct((B * N, DIM), jnp.float32),
            jax.ShapeDtypeStruct((B * N, DIMP), jnp.int32),
        ),
    )(feats2, xyzp, g1b1, wqa, wg)


def _gather_call(gtab, idxf_b):
    k = functools.partial(
        pl.kernel,
        mesh=plsc.VectorSubcoreMesh(core_axis_name="c", subcore_axis_name="s"),
        out_type=jax.ShapeDtypeStruct((NW, NPW, DIMP), jnp.int32),
        scratch_types=[
            pltpu.VMEM((NPW,), jnp.int32),
            pltpu.VMEM((CH, DIMP), jnp.int32),
            pltpu.VMEM((CH, DIMP), jnp.int32),
            pltpu.SemaphoreType.DMA,
            pltpu.SemaphoreType.DMA,
            pltpu.SemaphoreType.DMA,
            pltpu.SemaphoreType.DMA,
        ],
    )(_gather_body)
    return k(gtab, idxf_b)


def _attn_call(b, feats2, xyzp, qa, gg_b, wp1p, wp2cat, wa2, wproj, wf1, wf2,
               g2b2):
    row = lambda i, b=b: (b * NB + i, 0)
    return pl.pallas_call(
        _attn_body,
        grid=(NB,),
        in_specs=[
            pl.BlockSpec((BN, DIM), row),
            pl.BlockSpec((BN, XP), row),
            pl.BlockSpec((BN, DIM), row),
            pl.BlockSpec((1, K, BN, DIMP), lambda i: (0, 0, i, 0)),
            pl.BlockSpec((XP, DIM), lambda i: (0, 0)),
            pl.BlockSpec((DIM, DIM2), lambda i: (0, 0)),
            pl.BlockSpec((DIM, DIM), lambda i: (0, 0)),
            pl.BlockSpec((DIM, DIM), lambda i: (0, 0)),
            pl.BlockSpec((DIM, HID), lambda i: (0, 0)),
            pl.BlockSpec((HID, DIM), lambda i: (0, 0)),
            pl.BlockSpec((2, DIM), lambda i: (0, 0)),
        ],
        out_specs=pl.BlockSpec((BN, DIM), lambda i: (i, 0)),
        out_shape=jax.ShapeDtypeStruct((N, DIM), jnp.float32),
    )(feats2, xyzp, qa, gg_b, wp1p, wp2cat, wa2, wproj, wf1, wf2, g2b2)


def kernel(xyz, feats, idx, g1, b1, g2, b2, Wq, Wkv, Wp1, Wp2, Wa1, Wa2, Wproj, Wf1, Wf2):
    feats2 = feats.reshape(B * N, DIM)
    xyzp = jnp.pad(xyz, ((0, 0), (0, 0), (0, XP - 3))).reshape(B * N, XP)
    wp1p = jnp.pad(Wp1, ((0, XP - 3), (0, 0)))
    g1b1 = jnp.stack([g1, b1])
    g2b2 = jnp.stack([g2, b2])
    idxf = (idx + (jnp.arange(B, dtype=jnp.int32) * N)[:, None, None]
            ).transpose(0, 2, 1).reshape(B, NW, NPW)

    wqa, wg, wp2cat = _fold_call(Wq, Wa1, Wkv, Wp2)
    qa, gtab = _pre_call(feats2, xyzp, g1b1, wqa, wg)
    zs = []
    for b in range(B):
        gg_b = _gather_call(gtab, idxf[b]).reshape(1, K, N, DIMP)
        zs.append(_attn_call(b, feats2, xyzp, qa, gg_b,
                             wp1p, wp2cat, Wa2, Wproj, Wf1, Wf2, g2b2))
    return jnp.stack(zs)


# speedup vs baseline: 1.1420x; 1.0517x over previous
import functools

import jax
import jax.numpy as jnp
from jax import lax
from jax.experimental import pallas as pl
from jax.experimental.pallas import tpu as pltpu
from jax.experimental.pallas import tpu_sc as plsc

B, N, K, DIM, HID = 2, 4096, 16, 256, 512
DIM2 = 2 * DIM
XP = 128
DIMP = DIM + XP
BN = 256
NB = N // BN
BM = 1024
CH = 128
NSC, NSUB = 2, 16
NW = NSC * NSUB
PH = 2
NH = N // PH
NPW = (K * NH) // NW
CPW = NPW // CH


def _fold_body(wq, wa1, wkv, wp2, wqa_ref, wg_ref, wp2cat_ref):
    a1 = wa1[...]
    wqa_ref[...] = jnp.dot(wq[...], a1, preferred_element_type=jnp.float32)
    wg_ref[:, :DIM] = jnp.dot(wkv[:, :DIM], a1, preferred_element_type=jnp.float32)
    wg_ref[:, DIM:] = wkv[:, DIM:]
    wp2cat_ref[:, :DIM] = wp2[...]
    wp2cat_ref[:, DIM:] = jnp.dot(wp2[...], a1, preferred_element_type=jnp.float32)


def _rne16(x):
    u = lax.bitcast_convert_type(x, jnp.int32)
    return (u + 0x7FFF + ((u >> 16) & 1)) >> 16


def _pre_body(feats, xyzp, g1b1, wqa, wg, qa_ref, g_ref):
    x = feats[...]
    m = jnp.mean(x, axis=-1, keepdims=True)
    xc = x - m
    v = jnp.mean(xc * xc, axis=-1, keepdims=True)
    xn = xc * lax.rsqrt(v + 1e-5) * g1b1[0:1, :] + g1b1[1:2, :]
    qa_ref[...] = jnp.dot(xn, wqa[...], preferred_element_type=jnp.float32)
    kav = jnp.dot(xn, wg[...], preferred_element_type=jnp.float32)
    ka_b = _rne16(kav[:, :DIM]) & 0xFFFF
    v_b = _rne16(kav[:, DIM:])
    g_ref[:, :DIM] = ka_b | (v_b << 16)
    g_ref[:, DIM:] = lax.bitcast_convert_type(xyzp[...], jnp.int32)


def _gather_body(gtab, idxf, gg_out, idx_v, gbuf0, gbuf1, sem_g0, sem_g1,
                 sem_w0, sem_w1):
    wid = lax.axis_index("s") * NSC + lax.axis_index("c")
    pltpu.sync_copy(idxf.at[wid], idx_v)
    gbufs = (gbuf0, gbuf1)
    gsems = (sem_g0, sem_g1)
    wsems = (sem_w0, sem_w1)

    def start_gather(c):
        isl = idx_v.at[pl.ds(c * CH, CH)]
        return pltpu.async_copy(gtab.at[isl], gbufs[c % 2], gsems[c % 2])

    gathers = {0: start_gather(0)}
    writes = {}
    for c in range(CPW):
        b = c % 2
        if c + 1 < CPW:
            if c >= 1:
                writes[c - 1].wait()
            gathers[c + 1] = start_gather(c + 1)
        gathers[c].wait()
        writes[c] = pltpu.async_copy(gbufs[b], gg_out.at[wid, pl.ds(c * CH, CH)],
                                     wsems[b])
    writes[CPW - 2].wait()
    writes[CPW - 1].wait()


def _attn_body(feats, xyzp, qa, gg, wp1p, wp2cat, wa2, wproj, wf1, wf2,
               g2b2, out_ref):
    bf = jnp.bfloat16
    f = feats[...]
    xp = xyzp[...]
    q = qa[...]
    w1b = wp1p[...].astype(bf)
    w2b = wp2cat[...].astype(bf)
    wab = wa2[...].astype(bf)
    m = jnp.full((BN, DIM), -1e30, jnp.float32)
    s = jnp.zeros((BN, DIM), jnp.float32)
    acc = jnp.zeros((BN, DIM), jnp.float32)
    for k in range(K):
        slab = gg[0, k]
        kav = slab[:, :DIM]
        ka = lax.bitcast_convert_type(kav << 16, jnp.float32)
        v = lax.bitcast_convert_type(kav & jnp.int32(-65536), jnp.float32)
        rel = lax.bitcast_convert_type(slab[:, DIM:], jnp.float32) - xp
        h = jnp.maximum(jnp.dot(rel.astype(bf), w1b,
                                preferred_element_type=jnp.float32), 0.0)
        pp = jnp.dot(h.astype(bf), w2b, preferred_element_type=jnp.float32)
        t = jnp.dot(jnp.maximum(q - ka + pp[:, DIM:], 0.0).astype(bf), wab,
                    preferred_element_type=jnp.float32)
        mn = jnp.maximum(m, t)
        sc = jnp.exp(m - mn)
        e = jnp.exp(t - mn)
        s = s * sc + e
        acc = acc * sc + e * (v + pp[:, :DIM])
        m = mn
    out = acc / s
    y = f + jnp.dot(out.astype(bf), wproj[...].astype(bf),
                    preferred_element_type=jnp.float32)
    mu = jnp.mean(y, axis=-1, keepdims=True)
    yc = y - mu
    var = jnp.mean(yc * yc, axis=-1, keepdims=True)
    ln = yc * lax.rsqrt(var + 1e-5) * g2b2[0:1, :] + g2b2[1:2, :]
    z = y + jnp.dot(
        jnp.maximum(jnp.dot(ln.astype(bf), wf1[...].astype(bf),
                            preferred_element_type=jnp.float32), 0.0).astype(bf),
        wf2[...].astype(bf), preferred_element_type=jnp.float32)
    out_ref[...] = z


def _fold_call(wq, wa1, wkv, wp2):
    return pl.pallas_call(
        _fold_body,
        out_shape=(
            jax.ShapeDtypeStruct((DIM, DIM), jnp.float32),
            jax.ShapeDtypeStruct((DIM, DIM2), jnp.float32),
            jax.ShapeDtypeStruct((DIM, DIM2), jnp.float32),
        ),
    )(wq, wa1, wkv, wp2)


def _pre_call(feats2, xyzp, g1b1, wqa, wg):
    nblk = (B * N) // BM
    return pl.pallas_call(
        _pre_body,
        grid=(nblk,),
        in_specs=[
            pl.BlockSpec((BM, DIM), lambda i: (i, 0)),
            pl.BlockSpec((BM, XP), lambda i: (i, 0)),
            pl.BlockSpec((2, DIM), lambda i: (0, 0)),
            pl.BlockSpec((DIM, DIM), lambda i: (0, 0)),
            pl.BlockSpec((DIM, DIM2), lambda i: (0, 0)),
        ],
        out_specs=(
            pl.BlockSpec((BM, DIM), lambda i: (i, 0)),
            pl.BlockSpec((BM, DIMP), lambda i: (i, 0)),
        ),
        out_shape=(
            jax.ShapeDtypeStruct((B * N, DIM), jnp.float32),
            jax.ShapeDtypeStruct((B * N, DIMP), jnp.int32),
        ),
    )(feats2, xyzp, g1b1, wqa, wg)


def _gather_call(gtab, idxf_b):
    k = functools.partial(
        pl.kernel,
        mesh=plsc.VectorSubcoreMesh(core_axis_name="c", subcore_axis_name="s"),
        out_type=jax.ShapeDtypeStruct((NW, NPW, DIMP), jnp.int32),
        scratch_types=[
            pltpu.VMEM((NPW,), jnp.int32),
            pltpu.VMEM((CH, DIMP), jnp.int32),
            pltpu.VMEM((CH, DIMP), jnp.int32),
            pltpu.SemaphoreType.DMA,
            pltpu.SemaphoreType.DMA,
            pltpu.SemaphoreType.DMA,
            pltpu.SemaphoreType.DMA,
        ],
    )(_gather_body)
    return k(gtab, idxf_b)


def _attn_call(base, feats2, xyzp, qa, gg_p, wp1p, wp2cat, wa2, wproj, wf1,
               wf2, g2b2):
    nbh = NH // BN
    row = lambda i, base=base: (base + i, 0)
    return pl.pallas_call(
        _attn_body,
        grid=(nbh,),
        in_specs=[
            pl.BlockSpec((BN, DIM), row),
            pl.BlockSpec((BN, XP), row),
            pl.BlockSpec((BN, DIM), row),
            pl.BlockSpec((1, K, BN, DIMP), lambda i: (0, 0, i, 0)),
            pl.BlockSpec((XP, DIM), lambda i: (0, 0)),
            pl.BlockSpec((DIM, DIM2), lambda i: (0, 0)),
            pl.BlockSpec((DIM, DIM), lambda i: (0, 0)),
            pl.BlockSpec((DIM, DIM), lambda i: (0, 0)),
            pl.BlockSpec((DIM, HID), lambda i: (0, 0)),
            pl.BlockSpec((HID, DIM), lambda i: (0, 0)),
            pl.BlockSpec((2, DIM), lambda i: (0, 0)),
        ],
        out_specs=pl.BlockSpec((BN, DIM), lambda i: (i, 0)),
        out_shape=jax.ShapeDtypeStruct((NH, DIM), jnp.float32),
    )(feats2, xyzp, qa, gg_p, wp1p, wp2cat, wa2, wproj, wf1, wf2, g2b2)


def kernel(xyz, feats, idx, g1, b1, g2, b2, Wq, Wkv, Wp1, Wp2, Wa1, Wa2, Wproj, Wf1, Wf2):
    feats2 = feats.reshape(B * N, DIM)
    xyzp = jnp.pad(xyz, ((0, 0), (0, 0), (0, XP - 3))).reshape(B * N, XP)
    wp1p = jnp.pad(Wp1, ((0, XP - 3), (0, 0)))
    g1b1 = jnp.stack([g1, b1])
    g2b2 = jnp.stack([g2, b2])
    idxf = (idx + (jnp.arange(B, dtype=jnp.int32) * N)[:, None, None]
            ).transpose(0, 2, 1).reshape(B, K, PH, NH).transpose(0, 2, 1, 3
            ).reshape(B, PH, NW, NPW)

    wqa, wg, wp2cat = _fold_call(Wq, Wa1, Wkv, Wp2)
    qa, gtab = _pre_call(feats2, xyzp, g1b1, wqa, wg)
    zs = []
    for b in range(B):
        for h in range(PH):
            gg_p = _gather_call(gtab, idxf[b, h]).reshape(1, K, NH, DIMP)
            base = b * NB + h * (NH // BN)
            zs.append(_attn_call(base, feats2, xyzp, qa, gg_p,
                                 wp1p, wp2cat, Wa2, Wproj, Wf1, Wf2, g2b2))
    return jnp.concatenate(zs).reshape(B, N, DIM)


# speedup vs baseline: 1.1590x; 1.0148x over previous
import functools

import jax
import jax.numpy as jnp
from jax import lax
from jax.experimental import pallas as pl
from jax.experimental.pallas import tpu as pltpu
from jax.experimental.pallas import tpu_sc as plsc

B, N, K, DIM, HID = 2, 4096, 16, 256, 512
DIM2 = 2 * DIM
XP = 128
DIMP = DIM + XP
BN = 256
NB = N // BN
BM = 1024
CH = 128
NSC, NSUB = 2, 16
NW = NSC * NSUB
PH = 4
NH = N // PH
NPW = (K * NH) // NW
CPW = NPW // CH


def _fold_body(wq, wa1, wkv, wp2, wqa_ref, wg_ref, wp2cat_ref):
    a1 = wa1[...]
    wqa_ref[...] = jnp.dot(wq[...], a1, preferred_element_type=jnp.float32)
    wg_ref[:, :DIM] = jnp.dot(wkv[:, :DIM], a1, preferred_element_type=jnp.float32)
    wg_ref[:, DIM:] = wkv[:, DIM:]
    wp2cat_ref[:, :DIM] = wp2[...]
    wp2cat_ref[:, DIM:] = jnp.dot(wp2[...], a1, preferred_element_type=jnp.float32)


def _rne16(x):
    u = lax.bitcast_convert_type(x, jnp.int32)
    return (u + 0x7FFF + ((u >> 16) & 1)) >> 16


def _pre_body(feats, xyzp, g1b1, wqa, wg, qa_ref, g_ref):
    x = feats[...]
    m = jnp.mean(x, axis=-1, keepdims=True)
    xc = x - m
    v = jnp.mean(xc * xc, axis=-1, keepdims=True)
    xn = xc * lax.rsqrt(v + 1e-5) * g1b1[0:1, :] + g1b1[1:2, :]
    qa_ref[...] = jnp.dot(xn, wqa[...], preferred_element_type=jnp.float32)
    kav = jnp.dot(xn, wg[...], preferred_element_type=jnp.float32)
    ka_b = _rne16(kav[:, :DIM]) & 0xFFFF
    v_b = _rne16(kav[:, DIM:])
    g_ref[:, :DIM] = ka_b | (v_b << 16)
    g_ref[:, DIM:] = lax.bitcast_convert_type(xyzp[...], jnp.int32)


def _gather_body(gtab, idxf, gg_out, idx_v, gbuf0, gbuf1, sem_g0, sem_g1,
                 sem_w0, sem_w1):
    wid = lax.axis_index("s") * NSC + lax.axis_index("c")
    pltpu.sync_copy(idxf.at[wid], idx_v)
    gbufs = (gbuf0, gbuf1)
    gsems = (sem_g0, sem_g1)
    wsems = (sem_w0, sem_w1)

    def start_gather(c):
        isl = idx_v.at[pl.ds(c * CH, CH)]
        return pltpu.async_copy(gtab.at[isl], gbufs[c % 2], gsems[c % 2])

    gathers = {0: start_gather(0)}
    writes = {}
    for c in range(CPW):
        b = c % 2
        if c + 1 < CPW:
            if c >= 1:
                writes[c - 1].wait()
            gathers[c + 1] = start_gather(c + 1)
        gathers[c].wait()
        writes[c] = pltpu.async_copy(gbufs[b], gg_out.at[wid, pl.ds(c * CH, CH)],
                                     wsems[b])
    writes[CPW - 2].wait()
    writes[CPW - 1].wait()


def _attn_body(feats, xyzp, qa, gg, wp1p, wp2cat, wa2, wproj, wf1, wf2,
               g2b2, out_ref):
    bf = jnp.bfloat16
    f = feats[...]
    xp = xyzp[...]
    q = qa[...]
    w1b = wp1p[...].astype(bf)
    w2b = wp2cat[...].astype(bf)
    wab = wa2[...].astype(bf)
    m = jnp.full((BN, DIM), -1e30, jnp.float32)
    s = jnp.zeros((BN, DIM), jnp.float32)
    acc = jnp.zeros((BN, DIM), jnp.float32)
    for k in range(K):
        slab = gg[0, k]
        kav = slab[:, :DIM]
        ka = lax.bitcast_convert_type(kav << 16, jnp.float32)
        v = lax.bitcast_convert_type(kav & jnp.int32(-65536), jnp.float32)
        rel = lax.bitcast_convert_type(slab[:, DIM:], jnp.float32) - xp
        h = jnp.maximum(jnp.dot(rel.astype(bf), w1b,
                                preferred_element_type=jnp.float32), 0.0)
        pp = jnp.dot(h.astype(bf), w2b, preferred_element_type=jnp.float32)
        t = jnp.dot(jnp.maximum(q - ka + pp[:, DIM:], 0.0).astype(bf), wab,
                    preferred_element_type=jnp.float32)
        mn = jnp.maximum(m, t)
        sc = jnp.exp(m - mn)
        e = jnp.exp(t - mn)
        s = s * sc + e
        acc = acc * sc + e * (v + pp[:, :DIM])
        m = mn
    out = acc / s
    y = f + jnp.dot(out.astype(bf), wproj[...].astype(bf),
                    preferred_element_type=jnp.float32)
    mu = jnp.mean(y, axis=-1, keepdims=True)
    yc = y - mu
    var = jnp.mean(yc * yc, axis=-1, keepdims=True)
    ln = yc * lax.rsqrt(var + 1e-5) * g2b2[0:1, :] + g2b2[1:2, :]
    z = y + jnp.dot(
        jnp.maximum(jnp.dot(ln.astype(bf), wf1[...].astype(bf),
                            preferred_element_type=jnp.float32), 0.0).astype(bf),
        wf2[...].astype(bf), preferred_element_type=jnp.float32)
    out_ref[...] = z


def _fold_call(wq, wa1, wkv, wp2):
    return pl.pallas_call(
        _fold_body,
        out_shape=(
            jax.ShapeDtypeStruct((DIM, DIM), jnp.float32),
            jax.ShapeDtypeStruct((DIM, DIM2), jnp.float32),
            jax.ShapeDtypeStruct((DIM, DIM2), jnp.float32),
        ),
    )(wq, wa1, wkv, wp2)


def _pre_call(feats2, xyzp, g1b1, wqa, wg):
    nblk = (B * N) // BM
    return pl.pallas_call(
        _pre_body,
        grid=(nblk,),
        in_specs=[
            pl.BlockSpec((BM, DIM), lambda i: (i, 0)),
            pl.BlockSpec((BM, XP), lambda i: (i, 0)),
            pl.BlockSpec((2, DIM), lambda i: (0, 0)),
            pl.BlockSpec((DIM, DIM), lambda i: (0, 0)),
            pl.BlockSpec((DIM, DIM2), lambda i: (0, 0)),
        ],
        out_specs=(
            pl.BlockSpec((BM, DIM), lambda i: (i, 0)),
            pl.BlockSpec((BM, DIMP), lambda i: (i, 0)),
        ),
        out_shape=(
            jax.ShapeDtypeStruct((B * N, DIM), jnp.float32),
            jax.ShapeDtypeStruct((B * N, DIMP), jnp.int32),
        ),
    )(feats2, xyzp, g1b1, wqa, wg)


def _gather_call(gtab, idxf_b):
    k = functools.partial(
        pl.kernel,
        mesh=plsc.VectorSubcoreMesh(core_axis_name="c", subcore_axis_name="s"),
        out_type=jax.ShapeDtypeStruct((NW, NPW, DIMP), jnp.int32),
        scratch_types=[
            pltpu.VMEM((NPW,), jnp.int32),
            pltpu.VMEM((CH, DIMP), jnp.int32),
            pltpu.VMEM((CH, DIMP), jnp.int32),
            pltpu.SemaphoreType.DMA,
            pltpu.SemaphoreType.DMA,
            pltpu.SemaphoreType.DMA,
            pltpu.SemaphoreType.DMA,
        ],
    )(_gather_body)
    return k(gtab, idxf_b)


def _attn_call(base, feats2, xyzp, qa, gg_p, wp1p, wp2cat, wa2, wproj, wf1,
               wf2, g2b2):
    nbh = NH // BN
    row = lambda i, base=base: (base + i, 0)
    return pl.pallas_call(
        _attn_body,
        grid=(nbh,),
        in_specs=[
            pl.BlockSpec((BN, DIM), row),
            pl.BlockSpec((BN, XP), row),
            pl.BlockSpec((BN, DIM), row),
            pl.BlockSpec((1, K, BN, DIMP), lambda i: (0, 0, i, 0)),
            pl.BlockSpec((XP, DIM), lambda i: (0, 0)),
            pl.BlockSpec((DIM, DIM2), lambda i: (0, 0)),
            pl.BlockSpec((DIM, DIM), lambda i: (0, 0)),
            pl.BlockSpec((DIM, DIM), lambda i: (0, 0)),
            pl.BlockSpec((DIM, HID), lambda i: (0, 0)),
            pl.BlockSpec((HID, DIM), lambda i: (0, 0)),
            pl.BlockSpec((2, DIM), lambda i: (0, 0)),
        ],
        out_specs=pl.BlockSpec((BN, DIM), lambda i: (i, 0)),
        out_shape=jax.ShapeDtypeStruct((NH, DIM), jnp.float32),
    )(feats2, xyzp, qa, gg_p, wp1p, wp2cat, wa2, wproj, wf1, wf2, g2b2)


def kernel(xyz, feats, idx, g1, b1, g2, b2, Wq, Wkv, Wp1, Wp2, Wa1, Wa2, Wproj, Wf1, Wf2):
    feats2 = feats.reshape(B * N, DIM)
    xyzp = jnp.pad(xyz, ((0, 0), (0, 0), (0, XP - 3))).reshape(B * N, XP)
    wp1p = jnp.pad(Wp1, ((0, XP - 3), (0, 0)))
    g1b1 = jnp.stack([g1, b1])
    g2b2 = jnp.stack([g2, b2])
    idxf = (idx + (jnp.arange(B, dtype=jnp.int32) * N)[:, None, None]
            ).transpose(0, 2, 1).reshape(B, K, PH, NH).transpose(0, 2, 1, 3
            ).reshape(B, PH, NW, NPW)

    wqa, wg, wp2cat = _fold_call(Wq, Wa1, Wkv, Wp2)
    qa, gtab = _pre_call(feats2, xyzp, g1b1, wqa, wg)
    zs = []
    for b in range(B):
        for h in range(PH):
            gg_p = _gather_call(gtab, idxf[b, h]).reshape(1, K, NH, DIMP)
            base = b * NB + h * (NH // BN)
            zs.append(_attn_call(base, feats2, xyzp, qa, gg_p,
                                 wp1p, wp2cat, Wa2, Wproj, Wf1, Wf2, g2b2))
    return jnp.concatenate(zs).reshape(B, N, DIM)
